# baseline (speedup 1.0000x reference)
import functools

import jax
import jax.numpy as jnp
from jax import lax
from jax.experimental import pallas as pl
from jax.experimental.pallas import tpu as pltpu
from jax.experimental.pallas import tpu_sc as plsc

NH = 16
MAXD = 32
S = 2048
W = 2 * MAXD + 1
MPAD = 4224
GR = 8
K = 8


def _build_body(table_ref, m8_ref):
    v = jax.lax.broadcasted_iota(jnp.int32, (GR, MPAD), 1)
    r = jax.lax.broadcasted_iota(jnp.int32, (GR, MPAD), 0)
    idx = jnp.clip(v - r - S, -MAXD, MAXD) + MAXD
    acc = jnp.full((GR, MPAD), table_ref[0, 0, 0], dtype=jnp.float32)
    for k in range(1, W):
        acc = jnp.where(idx == k, table_ref[0, 0, k], acc)
    m8_ref[0] = acc


def _build_m8(relative_bias):
    return pl.pallas_call(
        _build_body,
        grid=(NH,),
        in_specs=[
            pl.BlockSpec((1, 1, W), lambda h: (h, 0, 0),
                         memory_space=pltpu.SMEM),
        ],
        out_specs=pl.BlockSpec((1, GR, MPAD), lambda h: (h, 0, 0)),
        out_shape=jax.ShapeDtypeStruct((NH, GR, MPAD), jnp.float32),
    )(relative_bias.reshape(NH, 1, W))


def _sc_materialize(m8):
    info = plsc.get_sparse_core_info()
    nc, ns = info.num_cores, info.num_subcores
    rows_per_w = NH * S // (nc * ns)
    ngroups = rows_per_w // GR
    mesh = plsc.VectorSubcoreMesh(core_axis_name="c", subcore_axis_name="s")

    @functools.partial(
        pl.kernel,
        mesh=mesh,
        out_type=jax.ShapeDtypeStruct((NH * S, S), jnp.float32),
        scratch_types=[
            pltpu.VMEM((GR, MPAD), jnp.float32),
            pltpu.SemaphoreType.DMA,
        ],
        compiler_params=pltpu.CompilerParams(use_tc_tiling_on_sc=False),
    )
    def sc_k(m8_hbm, out_hbm, m_v, sem):
        wid = lax.axis_index("s") * nc + lax.axis_index("c")
        h = wid // (S // rows_per_w)
        i0 = (wid % (S // rows_per_w)) * rows_per_w
        pltpu.sync_copy(m8_hbm.at[h], m_v)

        def group_copy(g):
            i_g = i0 + GR * g
            q = S - i_g
            row_g = h * S + i_g
            return pltpu.make_async_copy(
                m_v.at[:, pl.ds(pl.multiple_of(q, 8), S)],
                out_hbm.at[pl.ds(pl.multiple_of(row_g, 8), GR), :],
                sem,
            )

        def chunk(g, carry):
            group_copy(g).start()

            @pl.when(g >= K)
            def _drain():
                group_copy(g - K).wait()

            return carry

        lax.fori_loop(0, ngroups, chunk, 0)
        for g in range(K):
            group_copy(ngroups - K + g).wait()

    return sc_k(m8)


def kernel(seq_len, relative_bias):
    del seq_len
    m8 = _build_m8(relative_bias)
    return _sc_materialize(m8).reshape(NH, S, S)


# speedup vs baseline: 2.3835x; 2.3835x over previous
import functools

import jax
import jax.numpy as jnp
from jax import lax
from jax.experimental import pallas as pl
from jax.experimental.pallas import tpu as pltpu
from jax.experimental.pallas import tpu_sc as plsc

NH = 16
MAXD = 32
S = 2048
W = 2 * MAXD + 1
NP = 128
MPAD = 4352
GR = 8
GPH = 8


def _build_body(table_ref, sl_ref):
    t = jax.lax.broadcasted_iota(jnp.int32, (1, MPAD), 1)
    idx = jnp.clip(t - S, -MAXD, MAXD) + MAXD
    acc = jnp.full((1, MPAD), table_ref[0, 0, 0], dtype=jnp.float32)
    for k in range(1, W):
        acc = jnp.where(idx == k, table_ref[0, 0, k], acc)
    bm = jnp.broadcast_to(acc, (NP, MPAD))
    sl_ref[0] = pltpu.roll(bm, 0, 1, stride=1, stride_axis=0)


def _build_slab(relative_bias):
    return pl.pallas_call(
        _build_body,
        grid=(NH,),
        in_specs=[
            pl.BlockSpec((1, 1, W), lambda h: (h, 0, 0),
                         memory_space=pltpu.SMEM),
        ],
        out_specs=pl.BlockSpec((1, NP, MPAD), lambda h: (h, 0, 0)),
        out_shape=jax.ShapeDtypeStruct((NH, NP, MPAD), jnp.float32),
    )(relative_bias.reshape(NH, 1, W))


def _sc_materialize(slab):
    info = plsc.get_sparse_core_info()
    nc, ns = info.num_cores, info.num_subcores
    assert nc == 2 and ns == 16
    mesh = plsc.VectorSubcoreMesh(core_axis_name="c", subcore_axis_name="s")

    @functools.partial(
        pl.kernel,
        mesh=mesh,
        out_type=jax.ShapeDtypeStruct((NH, S, S), jnp.float32),
        scratch_types=[
            pltpu.VMEM((2, GR, MPAD), jnp.float32),
            pltpu.SemaphoreType.DMA((2,)),
            pltpu.SemaphoreType.DMA((2,)),
        ],
    )
    def sc_k(sl_hbm, out_hbm, m_v, ssem, gsem):
        sid = lax.axis_index("s")
        cid = lax.axis_index("c")
        rb = pl.multiple_of(GR * sid, GR)

        def stage(h, sl):
            return pltpu.make_async_copy(
                sl_hbm.at[h, pl.ds(rb, GR), :], m_v.at[sl], ssem.at[sl])

        def group(h, m, sl):
            i_g = pl.multiple_of(GR * sid + NP * (2 * m + cid), GR)
            v0 = pl.multiple_of(S - NP * (2 * m + cid), NP)
            return pltpu.make_async_copy(
                m_v.at[sl, :, pl.ds(v0, S)],
                out_hbm.at[h, pl.ds(i_g, GR), :],
                gsem.at[sl],
            )

        stage(0, 0).start()

        def head_step(h, carry):
            sl = lax.rem(h, 2)
            stage(h, sl).wait()
            for m in range(GPH):
                group(h, m, sl).start()

            @pl.when(h + 1 < NH)
            def _next():
                nsl = lax.rem(h + 1, 2)

                @pl.when(h >= 1)
                def _drain_prev():
                    for m in range(GPH):
                        group(h - 1, m, nsl).wait()

                stage(h + 1, nsl).start()

            return carry

        lax.fori_loop(0, NH, head_step, 0)
        for m in range(GPH):
            group(NH - 2, m, lax.rem(NH - 2, 2)).wait()
        for m in range(GPH):
            group(NH - 1, m, lax.rem(NH - 1, 2)).wait()

    return sc_k(slab)


def kernel(seq_len, relative_bias):
    del seq_len
    return _sc_materialize(_build_slab(relative_bias))


# speedup vs baseline: 2.4218x; 1.0161x over previous
import functools

import jax
import jax.numpy as jnp
from jax import lax
from jax.experimental import pallas as pl
from jax.experimental.pallas import tpu as pltpu
from jax.experimental.pallas import tpu_sc as plsc

NH = 16
MAXD = 32
S = 2048
W = 2 * MAXD + 1
NP = 128
MPAD = 4096
GR = 8
GPH = 8


def _build_body(table_ref, sl_ref):
    t = jax.lax.broadcasted_iota(jnp.int32, (1, MPAD), 1)
    idx = jnp.clip(t - S, -MAXD, MAXD) + MAXD
    acc = jnp.full((1, MPAD), table_ref[0, 0, 0], dtype=jnp.float32)
    for k in range(1, W):
        acc = jnp.where(idx == k, table_ref[0, 0, k], acc)
    bm = jnp.broadcast_to(acc, (NP, MPAD))
    sl_ref[0] = pltpu.roll(bm, 0, 1, stride=1, stride_axis=0)


def _build_slab(relative_bias):
    return pl.pallas_call(
        _build_body,
        grid=(NH,),
        in_specs=[
            pl.BlockSpec((1, 1, W), lambda h: (h, 0, 0),
                         memory_space=pltpu.SMEM),
        ],
        out_specs=pl.BlockSpec((1, NP, MPAD), lambda h: (h, 0, 0)),
        out_shape=jax.ShapeDtypeStruct((NH, NP, MPAD), jnp.float32),
    )(relative_bias.reshape(NH, 1, W))


def _sc_materialize(slab):
    info = plsc.get_sparse_core_info()
    nc, ns = info.num_cores, info.num_subcores
    assert nc == 2 and ns == 16
    mesh = plsc.VectorSubcoreMesh(core_axis_name="c", subcore_axis_name="s")

    @functools.partial(
        pl.kernel,
        mesh=mesh,
        out_type=jax.ShapeDtypeStruct((NH, S, S), jnp.float32),
        scratch_types=[
            pltpu.VMEM((2, GR, MPAD), jnp.float32),
            pltpu.SemaphoreType.DMA((2,)),
            pltpu.SemaphoreType.DMA((2,)),
        ],
    )
    def sc_k(sl_hbm, out_hbm, m_v, ssem, gsem):
        sid = lax.axis_index("s")
        cid = lax.axis_index("c")
        rb = pl.multiple_of(GR * sid, GR)

        def stage(h, sl):
            return pltpu.make_async_copy(
                sl_hbm.at[h, pl.ds(rb, GR), :], m_v.at[sl], ssem.at[sl])

        def group(h, m, sl):
            i_g = pl.multiple_of(GR * sid + NP * (2 * m + cid), GR)
            v0 = pl.multiple_of(S - NP * (2 * m + cid), NP)
            return pltpu.make_async_copy(
                m_v.at[sl, :, pl.ds(v0, S)],
                out_hbm.at[h, pl.ds(i_g, GR), :],
                gsem.at[sl],
            )

        stage(0, 0).start()

        def head_step(h, carry):
            sl = lax.rem(h, 2)
            stage(h, sl).wait()
            for m in range(GPH):
                group(h, m, sl).start()

            @pl.when(h + 1 < NH)
            def _next():
                nsl = lax.rem(h + 1, 2)

                @pl.when(h >= 1)
                def _drain_prev():
                    for m in range(GPH):
                        group(h - 1, m, nsl).wait()

                stage(h + 1, nsl).start()

            return carry

        lax.fori_loop(0, NH, head_step, 0)
        for m in range(GPH):
            group(NH - 2, m, lax.rem(NH - 2, 2)).wait()
        for m in range(GPH):
            group(NH - 1, m, lax.rem(NH - 1, 2)).wait()

    return sc_k(slab)


def kernel(seq_len, relative_bias):
    del seq_len
    return _sc_materialize(_build_slab(relative_bias))


# speedup vs baseline: 2.8018x; 1.1569x over previous
import functools

import jax
import jax.numpy as jnp
from jax import lax
from jax.experimental import pallas as pl
from jax.experimental.pallas import tpu as pltpu
from jax.experimental.pallas import tpu_sc as plsc

NH = 16
MAXD = 32
S = 2048
W = 2 * MAXD + 1
NP = 128
MPAD = 4096
GR = 8
GPH = 16
HPC = NH // 2


def _build_body(table_ref, sl_ref):
    t = jax.lax.broadcasted_iota(jnp.int32, (1, MPAD), 1)
    idx = jnp.clip(t - S, -MAXD, MAXD) + MAXD
    acc = jnp.full((1, MPAD), table_ref[0, 0, 0], dtype=jnp.float32)
    for k in range(1, W):
        acc = jnp.where(idx == k, table_ref[0, 0, k], acc)
    bm = jnp.broadcast_to(acc, (NP, MPAD))
    sl_ref[0] = pltpu.roll(bm, 0, 1, stride=1, stride_axis=0)


def _build_slab(relative_bias):
    return pl.pallas_call(
        _build_body,
        grid=(NH,),
        in_specs=[
            pl.BlockSpec((1, 1, W), lambda h: (h, 0, 0),
                         memory_space=pltpu.SMEM),
        ],
        out_specs=pl.BlockSpec((1, NP, MPAD), lambda h: (h, 0, 0)),
        out_shape=jax.ShapeDtypeStruct((NH, NP, MPAD), jnp.float32),
    )(relative_bias.reshape(NH, 1, W))


def _sc_materialize(slab):
    info = plsc.get_sparse_core_info()
    nc, ns = info.num_cores, info.num_subcores
    assert nc == 2 and ns == 16
    mesh = plsc.VectorSubcoreMesh(core_axis_name="c", subcore_axis_name="s")

    @functools.partial(
        pl.kernel,
        mesh=mesh,
        out_type=jax.ShapeDtypeStruct((NH, S, S), jnp.float32),
        scratch_types=[
            pltpu.VMEM((2, GR, MPAD), jnp.float32),
            pltpu.SemaphoreType.DMA((2,)),
            pltpu.SemaphoreType.DMA((2,)),
        ],
    )
    def sc_k(sl_hbm, out_hbm, m_v, ssem, gsem):
        sid = lax.axis_index("s")
        cid = lax.axis_index("c")
        rb = pl.multiple_of(GR * sid, GR)

        def stage(hh, sl):
            return pltpu.make_async_copy(
                sl_hbm.at[HPC * cid + hh, pl.ds(rb, GR), :],
                m_v.at[sl], ssem.at[sl])

        def group(hh, m, sl):
            i_g = pl.multiple_of(GR * sid + NP * m, GR)
            v0 = pl.multiple_of(S - NP * m, NP)
            return pltpu.make_async_copy(
                m_v.at[sl, :, pl.ds(v0, S)],
                out_hbm.at[HPC * cid + hh, pl.ds(i_g, GR), :],
                gsem.at[sl],
            )

        stage(0, 0).start()

        def head_step(hh, carry):
            sl = lax.rem(hh, 2)
            stage(hh, sl).wait()
            for m in range(GPH):
                group(hh, m, sl).start()

            @pl.when(hh + 1 < HPC)
            def _next():
                nsl = lax.rem(hh + 1, 2)

                @pl.when(hh >= 1)
                def _drain_prev():
                    for m in range(GPH):
                        group(hh - 1, m, nsl).wait()

                stage(hh + 1, nsl).start()

            return carry

        lax.fori_loop(0, HPC, head_step, 0)
        for m in range(GPH):
            group(HPC - 2, m, lax.rem(HPC - 2, 2)).wait()
        for m in range(GPH):
            group(HPC - 1, m, lax.rem(HPC - 1, 2)).wait()

    return sc_k(slab)


def kernel(seq_len, relative_bias):
    del seq_len
    return _sc_materialize(_build_slab(relative_bias))


# speedup vs baseline: 2.8301x; 1.0101x over previous
import functools

import jax
import jax.numpy as jnp
from jax import lax
from jax.experimental import pallas as pl
from jax.experimental.pallas import tpu as pltpu
from jax.experimental.pallas import tpu_sc as plsc

NH = 16
MAXD = 32
S = 2048
W = 2 * MAXD + 1
NP = 128
MPAD = 4096
GR = 8
GPH = 16
HPC = NH // 2


def _build_body(table_ref, sl_ref):
    t = jax.lax.broadcasted_iota(jnp.int32, (1, MPAD), 1)
    idx = jnp.clip(t - S, -MAXD, MAXD) + MAXD
    acc = jnp.full((1, MPAD), table_ref[0, 0, 0], dtype=jnp.float32)
    for k in range(1, W):
        acc = jnp.where(idx == k, table_ref[0, 0, k], acc)
    bm = jnp.broadcast_to(acc, (NP, MPAD))
    sl_ref[0] = pltpu.roll(bm, 0, 1, stride=1, stride_axis=0)


def _build_slab(relative_bias):
    return pl.pallas_call(
        _build_body,
        grid=(NH,),
        in_specs=[
            pl.BlockSpec((1, 1, W), lambda h: (h, 0, 0),
                         memory_space=pltpu.SMEM),
        ],
        out_specs=pl.BlockSpec((1, NP, MPAD), lambda h: (h, 0, 0)),
        out_shape=jax.ShapeDtypeStruct((NH, NP, MPAD), jnp.float32),
    )(relative_bias.reshape(NH, 1, W))


def _sc_materialize(slab):
    info = plsc.get_sparse_core_info()
    nc, ns = info.num_cores, info.num_subcores
    assert nc == 2 and ns == 16
    mesh = plsc.VectorSubcoreMesh(core_axis_name="c", subcore_axis_name="s")

    @functools.partial(
        pl.kernel,
        mesh=mesh,
        out_type=jax.ShapeDtypeStruct((NH, S, S), jnp.float32),
        scratch_types=[
            pltpu.VMEM((3, GR, MPAD), jnp.float32),
            pltpu.SemaphoreType.DMA((3,)),
            pltpu.SemaphoreType.DMA((3,)),
        ],
    )
    def sc_k(sl_hbm, out_hbm, m_v, ssem, gsem):
        sid = lax.axis_index("s")
        cid = lax.axis_index("c")
        rb = pl.multiple_of(GR * sid, GR)

        def stage(hh, sl):
            return pltpu.make_async_copy(
                sl_hbm.at[HPC * cid + hh, pl.ds(rb, GR), :],
                m_v.at[sl], ssem.at[sl])

        def group(hh, m, sl):
            i_g = pl.multiple_of(GR * sid + NP * m, GR)
            v0 = pl.multiple_of(S - NP * m, NP)
            return pltpu.make_async_copy(
                m_v.at[sl, :, pl.ds(v0, S)],
                out_hbm.at[HPC * cid + hh, pl.ds(i_g, GR), :],
                gsem.at[sl],
            )

        stage(0, 0).start()
        stage(1, 1).start()

        def head_step(hh, carry):
            sl = lax.rem(hh, 3)
            stage(hh, sl).wait()
            for m in range(GPH):
                group(hh, m, sl).start()

            @pl.when(hh + 2 < HPC)
            def _next():
                nsl = lax.rem(hh + 2, 3)

                @pl.when(hh >= 1)
                def _drain_prev():
                    for m in range(GPH):
                        group(hh - 1, m, nsl).wait()

                stage(hh + 2, nsl).start()

            return carry

        lax.fori_loop(0, HPC, head_step, 0)
        for hh in range(HPC - 3, HPC):
            for m in range(GPH):
                group(hh, m, lax.rem(hh, 3)).wait()

    return sc_k(slab)


def kernel(seq_len, relative_bias):
    del seq_len
    return _sc_materialize(_build_slab(relative_bias))


# speedup vs baseline: 2.8488x; 1.0066x over previous
import functools

import jax
import jax.numpy as jnp
from jax import lax
from jax.experimental import pallas as pl
from jax.experimental.pallas import tpu as pltpu
from jax.experimental.pallas import tpu_sc as plsc

NH = 16
MAXD = 32
S = 2048
W = 2 * MAXD + 1
NP = 128
MPAD = 4096
GR = 8
GPH = 16
HPC = NH // 2


def _build_body(table_ref, sl_ref):
    t = jax.lax.broadcasted_iota(jnp.int32, (1, MPAD), 1)
    idx = jnp.clip(t - S, -MAXD, MAXD) + MAXD
    acc = jnp.full((1, MPAD), table_ref[0, 0, 0], dtype=jnp.float32)
    for k in range(1, W):
        acc = jnp.where(idx == k, table_ref[0, 0, k], acc)
    bm = jnp.broadcast_to(acc, (NP, MPAD))
    sl_ref[0] = pltpu.roll(bm, 0, 1, stride=1, stride_axis=0)


def _build_slab(relative_bias):
    return pl.pallas_call(
        _build_body,
        grid=(NH,),
        in_specs=[
            pl.BlockSpec((1, 1, W), lambda h: (h, 0, 0),
                         memory_space=pltpu.SMEM),
        ],
        out_specs=pl.BlockSpec((1, NP, MPAD), lambda h: (h, 0, 0)),
        out_shape=jax.ShapeDtypeStruct((NH, NP, MPAD), jnp.float32),
    )(relative_bias.reshape(NH, 1, W))


def _sc_materialize(slab):
    info = plsc.get_sparse_core_info()
    nc, ns = info.num_cores, info.num_subcores
    assert nc == 2 and ns == 16
    mesh = plsc.VectorSubcoreMesh(core_axis_name="c", subcore_axis_name="s")

    @functools.partial(
        pl.kernel,
        mesh=mesh,
        out_type=jax.ShapeDtypeStruct((NH, S, S), jnp.float32),
        scratch_types=[
            pltpu.VMEM((3, GR, MPAD), jnp.float32),
            pltpu.SemaphoreType.DMA((3,)),
            pltpu.SemaphoreType.DMA((3,)),
        ],
    )
    def sc_k(sl_hbm, out_hbm, m_v, ssem, gsem):
        sid = lax.axis_index("s")
        cid = lax.axis_index("c")
        rb = pl.multiple_of(GR * sid, GR)

        def stage(hh, sl):
            return pltpu.make_async_copy(
                sl_hbm.at[HPC * cid + hh, pl.ds(rb, GR), :],
                m_v.at[sl], ssem.at[sl])

        def group(hh, m, sl):
            i_g = pl.multiple_of(GR * sid + NP * m, GR)
            v0 = pl.multiple_of(S - NP * m, NP)
            return pltpu.make_async_copy(
                m_v.at[sl, :, pl.ds(v0, S)],
                out_hbm.at[HPC * cid + hh, pl.ds(i_g, GR), :],
                gsem.at[sl],
            )

        stage(0, 0).start()
        stage(1, 1).start()

        def head_step(hh, carry):
            sl = lax.rem(hh, 3)

            @pl.when(hh >= 1)
            def _drain_prev():
                for m in range(GPH):
                    group(hh - 1, m, lax.rem(hh - 1, 3)).wait()

            stage(hh, sl).wait()
            for m in range(GPH):
                group(hh, m, sl).start()

            @pl.when(hh + 2 < HPC)
            def _next():
                stage(hh + 2, lax.rem(hh + 2, 3)).start()

            return carry

        lax.fori_loop(0, HPC, head_step, 0)
        for m in range(GPH):
            group(HPC - 1, m, lax.rem(HPC - 1, 3)).wait()

    return sc_k(slab)


def kernel(seq_len, relative_bias):
    del seq_len
    return _sc_materialize(_build_slab(relative_bias))
